# baseline (speedup 1.0000x reference)
import functools

import jax
import jax.numpy as jnp
from jax import lax
from jax.experimental import pallas as pl
from jax.experimental.pallas import tpu as pltpu
from jax.experimental.pallas import tpu_sc as plsc

BATCH = 16384
DIM = 16

_info = plsc.get_sparse_core_info()
_NC, _NS = _info.num_cores, _info.num_subcores
_NW = _NC * _NS
_B_PER_W = BATCH // _NW


def _gmf_body(uid_hbm, iid_hbm, utab_hbm, itab_hbm, out_hbm,
              uidx_v, iidx_v, urow_v, irow_v, sem):
    wid = lax.axis_index("s") * _NC + lax.axis_index("c")
    base = wid * _B_PER_W
    pltpu.sync_copy(uid_hbm.at[pl.ds(base, _B_PER_W)], uidx_v)
    pltpu.sync_copy(iid_hbm.at[pl.ds(base, _B_PER_W)], iidx_v)
    cu = pltpu.async_copy(utab_hbm.at[uidx_v], urow_v, sem)
    ci = pltpu.async_copy(itab_hbm.at[iidx_v], irow_v, sem)
    cu.wait()
    ci.wait()

    def body(i, carry):
        urow_v[i] = urow_v[i] * irow_v[i]
        return carry

    lax.fori_loop(0, _B_PER_W, body, 0)
    pltpu.sync_copy(urow_v, out_hbm.at[pl.ds(base, _B_PER_W)])


@jax.jit
def kernel(user_id, item_id, user_embed, item_embed):
    f = pl.kernel(
        _gmf_body,
        out_type=jax.ShapeDtypeStruct((BATCH, DIM), jnp.float32),
        mesh=plsc.VectorSubcoreMesh(core_axis_name="c", subcore_axis_name="s"),
        compiler_params=pltpu.CompilerParams(use_tc_tiling_on_sc=False),
        scratch_types=[
            pltpu.VMEM((_B_PER_W,), jnp.int32),
            pltpu.VMEM((_B_PER_W,), jnp.int32),
            pltpu.VMEM((_B_PER_W, DIM), jnp.float32),
            pltpu.VMEM((_B_PER_W, DIM), jnp.float32),
            pltpu.SemaphoreType.DMA,
        ],
    )
    return f(user_id.astype(jnp.int32), item_id.astype(jnp.int32),
             user_embed, item_embed)
